# baseline (speedup 1.0000x reference)
import functools

import jax
import jax.numpy as jnp
from jax import lax
from jax.experimental import pallas as pl
from jax.experimental.pallas import tpu as pltpu
from jax.experimental.pallas import tpu_sc as plsc

_B, _D, _E, _K = 2048, 768, 8, 2
_SBLK = 256
_S = 6144
_M = _S // _SBLK
_NW = 32
_TPW = _B // _NW
_HTOK = _TPW // 2
_NV = _D // 16


def _router_kernel(x_ref, wrt_ref, rb_ref,
                   d1_ref, d2_ref, w1_ref, w2_ref, bexp_ref, loss_ref):
    x = x_ref[...]

    logits = jnp.dot(x, wrt_ref[...],
                     preferred_element_type=jnp.float32) + rb_ref[...]
    m = jnp.max(logits, axis=-1, keepdims=True)
    ex = jnp.exp(logits - m)
    probs = ex / jnp.sum(ex, axis=-1, keepdims=True)

    eidx = jax.lax.broadcasted_iota(jnp.int32, probs.shape, 1)
    a1 = jnp.argmax(probs, axis=-1)[:, None]
    oh1 = (eidx == a1).astype(jnp.float32)
    w1 = jnp.max(probs, axis=-1, keepdims=True)
    probs2 = jnp.where(oh1 > 0, -jnp.inf, probs)
    a2 = jnp.argmax(probs2, axis=-1)[:, None]
    oh2 = (eidx == a2).astype(jnp.float32)
    w2 = jnp.max(probs2, axis=-1, keepdims=True)

    denom = w1 + w2
    w1_ref[...] = w1 / denom
    w2_ref[...] = w2 / denom
    mask = oh1 + oh2

    r_iota = jax.lax.broadcasted_iota(jnp.int32, (_SBLK, _SBLK), 0)
    c_iota = jax.lax.broadcasted_iota(jnp.int32, (_SBLK, _SBLK), 1)
    tri = (c_iota < r_iota).astype(jnp.float32)
    nchunks = _B // _SBLK
    ranks = []
    off = jnp.zeros((1, _E), jnp.float32)
    for c in range(nchunks):
        mc = mask[c * _SBLK:(c + 1) * _SBLK, :]
        local = jnp.dot(tri, mc, preferred_element_type=jnp.float32)
        ranks.append(local + off)
        off = off + jnp.sum(mc, axis=0, keepdims=True)
    rank = jnp.concatenate(ranks, axis=0)
    counts_row = off

    nblk_row = jnp.floor((counts_row + (_SBLK - 1)) * (1.0 / _SBLK))
    e_r = jax.lax.broadcasted_iota(jnp.int32, (_E, _E), 0)
    e_c = jax.lax.broadcasted_iota(jnp.int32, (_E, _E), 1)
    u_excl = (e_r < e_c).astype(jnp.float32)
    starts = float(_SBLK) * jnp.dot(nblk_row, u_excl,
                                    preferred_element_type=jnp.float32)

    slot = rank + starts
    d1 = jnp.sum(oh1 * slot, axis=-1, keepdims=True)
    d2 = jnp.sum(oh2 * slot, axis=-1, keepdims=True)
    d1_ref[...] = d1.astype(jnp.int32)
    d2_ref[...] = d2.astype(jnp.int32)

    ones_col = jnp.ones((_B, 1), jnp.float32)
    counts_col = jax.lax.dot_general(
        mask, ones_col, dimension_numbers=(((0,), (0,)), ((), ())),
        preferred_element_type=jnp.float32)
    nblk_col = jnp.floor((counts_col + (_SBLK - 1)) * (1.0 / _SBLK))
    l_incl = (e_c <= e_r).astype(jnp.float32)
    incl_col = jnp.dot(l_incl, nblk_col,
                       preferred_element_type=jnp.float32)
    miov = jax.lax.broadcasted_iota(jnp.int32, (1, _M), 1).astype(jnp.float32)
    bexp = jnp.sum((miov >= incl_col).astype(jnp.float32), axis=0,
                   keepdims=True)
    bexp_ref[...] = jnp.minimum(bexp, float(_E - 1)).astype(jnp.int32)

    psum = jnp.sum(probs, axis=0, keepdims=True)
    loss_ref[...] = jnp.sum(psum * counts_row, keepdims=True) / (_B * _B)


def _gmm_kernel(s_ref, xs_ref, we_ref, be_ref, ys_ref):
    del s_ref
    y = jax.lax.dot_general(
        xs_ref[...], we_ref[0],
        dimension_numbers=(((1,), (1,)), ((), ())),
        preferred_element_type=jnp.float32)
    ys_ref[...] = y + be_ref[0]


def _dispatch_body(x_hbm, d1_hbm, d2_hbm, xs_hbm, d1_v, d2_v, rows_v):
    wid = lax.axis_index("s") * 2 + lax.axis_index("c")
    base = wid * _TPW
    pltpu.sync_copy(d1_hbm.at[pl.ds(base, _TPW)], d1_v)
    pltpu.sync_copy(d2_hbm.at[pl.ds(base, _TPW)], d2_v)
    pltpu.sync_copy(x_hbm.at[pl.ds(base, _TPW)], rows_v)
    pltpu.sync_copy(rows_v, xs_hbm.at[d1_v])
    pltpu.sync_copy(rows_v, xs_hbm.at[d2_v])


def _combine_body(ys_hbm, d1_hbm, d2_hbm, w1_hbm, w2_hbm, out_hbm,
                  d1h, d2h, w1h, w2h, r1_v, r2_v, out_v):
    wid = lax.axis_index("s") * 2 + lax.axis_index("c")
    for h in range(2):
        tbase = wid * _TPW + h * _HTOK
        pltpu.sync_copy(d1_hbm.at[pl.ds(tbase, _HTOK)], d1h)
        pltpu.sync_copy(d2_hbm.at[pl.ds(tbase, _HTOK)], d2h)
        pltpu.sync_copy(w1_hbm.at[pl.ds(tbase, _HTOK)], w1h.at[pl.ds(0, _HTOK)])
        pltpu.sync_copy(w2_hbm.at[pl.ds(tbase, _HTOK)], w2h.at[pl.ds(0, _HTOK)])
        pltpu.sync_copy(ys_hbm.at[d1h], r1_v)
        pltpu.sync_copy(ys_hbm.at[d2h], r2_v)

        def body(t, carry):
            w1s = w1h[pl.ds(t, 16)][0]
            w2s = w2h[pl.ds(t, 16)][0]
            for vi in range(_NV):
                sl = pl.ds(vi * 16, 16)
                out_v[t, sl] = w1s * r1_v[t, sl] + w2s * r2_v[t, sl]
            return carry

        lax.fori_loop(0, _HTOK, body, 0)
        pltpu.sync_copy(out_v, out_hbm.at[pl.ds(tbase, _HTOK)])


def kernel(x, context_vector, Wr, br, We, be, context_weight):
    rb = (br + context_weight * context_vector).reshape(1, _E)
    wrt = Wr.T

    d1, d2, w1, w2, bexp, loss = pl.pallas_call(
        _router_kernel,
        out_shape=[
            jax.ShapeDtypeStruct((_B, 1), jnp.int32),
            jax.ShapeDtypeStruct((_B, 1), jnp.int32),
            jax.ShapeDtypeStruct((_B, 1), jnp.float32),
            jax.ShapeDtypeStruct((_B, 1), jnp.float32),
            jax.ShapeDtypeStruct((1, _M), jnp.int32),
            jax.ShapeDtypeStruct((1, 1), jnp.float32),
        ],
    )(x, wrt, rb)

    d1v = d1.reshape(_B)
    d2v = d2.reshape(_B)
    w1v = w1.reshape(_B)
    w2v = w2.reshape(_B)
    bexpv = bexp.reshape(_M)

    mesh = plsc.VectorSubcoreMesh(core_axis_name="c", subcore_axis_name="s")

    dispatch = functools.partial(
        pl.kernel, mesh=mesh,
        out_type=jax.ShapeDtypeStruct((_S, _D), jnp.float32),
        scratch_types=[
            pltpu.VMEM((_TPW,), jnp.int32),
            pltpu.VMEM((_TPW,), jnp.int32),
            pltpu.VMEM((_TPW, _D), jnp.float32),
        ],
    )(_dispatch_body)
    xs = dispatch(x, d1v, d2v)

    ys = pl.pallas_call(
        _gmm_kernel,
        grid_spec=pltpu.PrefetchScalarGridSpec(
            num_scalar_prefetch=1,
            grid=(_M,),
            in_specs=[
                pl.BlockSpec((_SBLK, _D), lambda i, s: (i, 0)),
                pl.BlockSpec((1, _D, _D), lambda i, s: (s[i], 0, 0)),
                pl.BlockSpec((1, 1, _D), lambda i, s: (s[i], 0, 0)),
            ],
            out_specs=pl.BlockSpec((_SBLK, _D), lambda i, s: (i, 0)),
        ),
        out_shape=jax.ShapeDtypeStruct((_S, _D), jnp.float32),
    )(bexpv, xs, We, be.reshape(_E, 1, _D))

    combine = functools.partial(
        pl.kernel, mesh=mesh,
        out_type=jax.ShapeDtypeStruct((_B, _D), jnp.float32),
        scratch_types=[
            pltpu.VMEM((_HTOK,), jnp.int32),
            pltpu.VMEM((_HTOK,), jnp.int32),
            pltpu.VMEM((_HTOK + 16,), jnp.float32),
            pltpu.VMEM((_HTOK + 16,), jnp.float32),
            pltpu.VMEM((_HTOK, _D), jnp.float32),
            pltpu.VMEM((_HTOK, _D), jnp.float32),
            pltpu.VMEM((_HTOK, _D), jnp.float32),
        ],
    )(_combine_body)
    out = combine(ys, d1v, d2v, w1v, w2v)

    return out, loss[0, 0]


# speedup vs baseline: 1.0762x; 1.0762x over previous
import functools

import jax
import jax.numpy as jnp
from jax import lax
from jax.experimental import pallas as pl
from jax.experimental.pallas import tpu as pltpu
from jax.experimental.pallas import tpu_sc as plsc

_B, _D, _E, _K = 2048, 768, 8, 2
_SBLK = 256
_S = 6144
_M = _S // _SBLK
_NW = 32
_TPW = _B // _NW
_HTOK = _TPW // 2
_NV = _D // 16


def _router_kernel(x_ref, wrt_ref, rb_ref,
                   d1_ref, d2_ref, w1_ref, w2_ref, bexp_ref, loss_ref):
    x = x_ref[...]

    logits = jnp.dot(x, wrt_ref[...],
                     preferred_element_type=jnp.float32) + rb_ref[...]
    m = jnp.max(logits, axis=-1, keepdims=True)
    ex = jnp.exp(logits - m)
    probs = ex / jnp.sum(ex, axis=-1, keepdims=True)

    eidx = jax.lax.broadcasted_iota(jnp.int32, probs.shape, 1)
    a1 = jnp.argmax(probs, axis=-1)[:, None]
    oh1 = (eidx == a1).astype(jnp.float32)
    w1 = jnp.max(probs, axis=-1, keepdims=True)
    probs2 = jnp.where(oh1 > 0, -jnp.inf, probs)
    a2 = jnp.argmax(probs2, axis=-1)[:, None]
    oh2 = (eidx == a2).astype(jnp.float32)
    w2 = jnp.max(probs2, axis=-1, keepdims=True)

    denom = w1 + w2
    w1_ref[...] = w1 / denom
    w2_ref[...] = w2 / denom
    mask = oh1 + oh2

    r_iota = jax.lax.broadcasted_iota(jnp.int32, (_SBLK, _SBLK), 0)
    c_iota = jax.lax.broadcasted_iota(jnp.int32, (_SBLK, _SBLK), 1)
    tri = (c_iota < r_iota).astype(jnp.float32)
    nchunks = _B // _SBLK
    ranks = []
    off = jnp.zeros((1, _E), jnp.float32)
    for c in range(nchunks):
        mc = mask[c * _SBLK:(c + 1) * _SBLK, :]
        local = jnp.dot(tri, mc, preferred_element_type=jnp.float32)
        ranks.append(local + off)
        off = off + jnp.sum(mc, axis=0, keepdims=True)
    rank = jnp.concatenate(ranks, axis=0)
    counts_row = off

    nblk_row = jnp.floor((counts_row + (_SBLK - 1)) * (1.0 / _SBLK))
    e_r = jax.lax.broadcasted_iota(jnp.int32, (_E, _E), 0)
    e_c = jax.lax.broadcasted_iota(jnp.int32, (_E, _E), 1)
    u_excl = (e_r < e_c).astype(jnp.float32)
    starts = float(_SBLK) * jnp.dot(nblk_row, u_excl,
                                    preferred_element_type=jnp.float32)

    slot = rank + starts
    d1 = jnp.sum(oh1 * slot, axis=-1, keepdims=True)
    d2 = jnp.sum(oh2 * slot, axis=-1, keepdims=True)
    d1_ref[...] = d1.astype(jnp.int32)
    d2_ref[...] = d2.astype(jnp.int32)

    ones_col = jnp.ones((_B, 1), jnp.float32)
    counts_col = jax.lax.dot_general(
        mask, ones_col, dimension_numbers=(((0,), (0,)), ((), ())),
        preferred_element_type=jnp.float32)
    nblk_col = jnp.floor((counts_col + (_SBLK - 1)) * (1.0 / _SBLK))
    l_incl = (e_c <= e_r).astype(jnp.float32)
    incl_col = jnp.dot(l_incl, nblk_col,
                       preferred_element_type=jnp.float32)
    miov = jax.lax.broadcasted_iota(jnp.int32, (1, _M), 1).astype(jnp.float32)
    bexp = jnp.sum((miov >= incl_col).astype(jnp.float32), axis=0,
                   keepdims=True)
    bexp_ref[...] = jnp.minimum(bexp, float(_E - 1)).astype(jnp.int32)

    psum = jnp.sum(probs, axis=0, keepdims=True)
    loss_ref[...] = jnp.sum(psum * counts_row, keepdims=True) / (_B * _B)


def _gmm_kernel(s_ref, xs_ref, we_ref, be_ref, ys_ref):
    del s_ref
    y = jax.lax.dot_general(
        xs_ref[...], we_ref[0],
        dimension_numbers=(((1,), (1,)), ((), ())),
        preferred_element_type=jnp.float32)
    ys_ref[...] = y + be_ref[0]


def _dispatch_body(x_hbm, d1_hbm, d2_hbm, xs_hbm, d1_v, d2_v, rows_v,
                   sem_ld, sem_st):
    wid = lax.axis_index("s") * 2 + lax.axis_index("c")
    base = wid * _TPW
    c1 = pltpu.async_copy(d1_hbm.at[pl.ds(base, _TPW)], d1_v, sem_ld)
    c2 = pltpu.async_copy(d2_hbm.at[pl.ds(base, _TPW)], d2_v, sem_ld)
    c3 = pltpu.async_copy(x_hbm.at[pl.ds(base, _TPW)], rows_v, sem_ld)
    c1.wait()
    c2.wait()
    c3.wait()
    s1 = pltpu.async_copy(rows_v, xs_hbm.at[d1_v], sem_st)
    s2 = pltpu.async_copy(rows_v, xs_hbm.at[d2_v], sem_st)
    s1.wait()
    s2.wait()


def _combine_body(ys_hbm, d1_hbm, d2_hbm, w1_hbm, w2_hbm, out_hbm,
                  d1a, d2a, d1b, d2b, w1a, w2a, w1b, w2b,
                  r1a, r2a, r1b, r2b, sem_i, sem_a, sem_b, sem_o):
    wid = lax.axis_index("s") * 2 + lax.axis_index("c")
    t0 = wid * _TPW

    ic = [
        pltpu.async_copy(d1_hbm.at[pl.ds(t0, _HTOK)], d1a, sem_i),
        pltpu.async_copy(d2_hbm.at[pl.ds(t0, _HTOK)], d2a, sem_i),
        pltpu.async_copy(d1_hbm.at[pl.ds(t0 + _HTOK, _HTOK)], d1b, sem_i),
        pltpu.async_copy(d2_hbm.at[pl.ds(t0 + _HTOK, _HTOK)], d2b, sem_i),
        pltpu.async_copy(w1_hbm.at[pl.ds(t0, _HTOK)],
                         w1a.at[pl.ds(0, _HTOK)], sem_i),
        pltpu.async_copy(w2_hbm.at[pl.ds(t0, _HTOK)],
                         w2a.at[pl.ds(0, _HTOK)], sem_i),
        pltpu.async_copy(w1_hbm.at[pl.ds(t0 + _HTOK, _HTOK)],
                         w1b.at[pl.ds(0, _HTOK)], sem_i),
        pltpu.async_copy(w2_hbm.at[pl.ds(t0 + _HTOK, _HTOK)],
                         w2b.at[pl.ds(0, _HTOK)], sem_i),
    ]
    for c in ic:
        c.wait()

    g1 = pltpu.async_copy(ys_hbm.at[d1a], r1a, sem_a)
    g2 = pltpu.async_copy(ys_hbm.at[d2a], r2a, sem_a)
    g3 = pltpu.async_copy(ys_hbm.at[d1b], r1b, sem_b)
    g4 = pltpu.async_copy(ys_hbm.at[d2b], r2b, sem_b)

    def mk_body(r1, r2, w1r, w2r):
        def body(t, carry):
            w1s = w1r[pl.ds(t, 16)][0]
            w2s = w2r[pl.ds(t, 16)][0]
            for vi in range(_NV):
                sl = pl.ds(vi * 16, 16)
                r1[t, sl] = w1s * r1[t, sl] + w2s * r2[t, sl]
            return carry
        return body

    g1.wait()
    g2.wait()
    lax.fori_loop(0, _HTOK, mk_body(r1a, r2a, w1a, w2a), 0)
    s1 = pltpu.async_copy(r1a, out_hbm.at[pl.ds(t0, _HTOK)], sem_o)

    g3.wait()
    g4.wait()
    lax.fori_loop(0, _HTOK, mk_body(r1b, r2b, w1b, w2b), 0)
    s2 = pltpu.async_copy(r1b, out_hbm.at[pl.ds(t0 + _HTOK, _HTOK)], sem_o)
    s1.wait()
    s2.wait()


def kernel(x, context_vector, Wr, br, We, be, context_weight):
    rb = (br + context_weight * context_vector).reshape(1, _E)
    wrt = Wr.T

    d1, d2, w1, w2, bexp, loss = pl.pallas_call(
        _router_kernel,
        out_shape=[
            jax.ShapeDtypeStruct((_B, 1), jnp.int32),
            jax.ShapeDtypeStruct((_B, 1), jnp.int32),
            jax.ShapeDtypeStruct((_B, 1), jnp.float32),
            jax.ShapeDtypeStruct((_B, 1), jnp.float32),
            jax.ShapeDtypeStruct((1, _M), jnp.int32),
            jax.ShapeDtypeStruct((1, 1), jnp.float32),
        ],
    )(x, wrt, rb)

    d1v = d1.reshape(_B)
    d2v = d2.reshape(_B)
    w1v = w1.reshape(_B)
    w2v = w2.reshape(_B)
    bexpv = bexp.reshape(_M)

    mesh = plsc.VectorSubcoreMesh(core_axis_name="c", subcore_axis_name="s")

    dispatch = functools.partial(
        pl.kernel, mesh=mesh,
        out_type=jax.ShapeDtypeStruct((_S, _D), jnp.float32),
        scratch_types=[
            pltpu.VMEM((_TPW,), jnp.int32),
            pltpu.VMEM((_TPW,), jnp.int32),
            pltpu.VMEM((_TPW, _D), jnp.float32),
            pltpu.SemaphoreType.DMA,
            pltpu.SemaphoreType.DMA,
        ],
    )(_dispatch_body)
    xs = dispatch(x, d1v, d2v)

    ys = pl.pallas_call(
        _gmm_kernel,
        grid_spec=pltpu.PrefetchScalarGridSpec(
            num_scalar_prefetch=1,
            grid=(_M,),
            in_specs=[
                pl.BlockSpec((_SBLK, _D), lambda i, s: (i, 0)),
                pl.BlockSpec((1, _D, _D), lambda i, s: (s[i], 0, 0)),
                pl.BlockSpec((1, 1, _D), lambda i, s: (s[i], 0, 0)),
            ],
            out_specs=pl.BlockSpec((_SBLK, _D), lambda i, s: (i, 0)),
        ),
        out_shape=jax.ShapeDtypeStruct((_S, _D), jnp.float32),
    )(bexpv, xs, We, be.reshape(_E, 1, _D))

    combine = functools.partial(
        pl.kernel, mesh=mesh,
        out_type=jax.ShapeDtypeStruct((_B, _D), jnp.float32),
        scratch_types=[
            pltpu.VMEM((_HTOK,), jnp.int32),
            pltpu.VMEM((_HTOK,), jnp.int32),
            pltpu.VMEM((_HTOK,), jnp.int32),
            pltpu.VMEM((_HTOK,), jnp.int32),
            pltpu.VMEM((_HTOK + 16,), jnp.float32),
            pltpu.VMEM((_HTOK + 16,), jnp.float32),
            pltpu.VMEM((_HTOK + 16,), jnp.float32),
            pltpu.VMEM((_HTOK + 16,), jnp.float32),
            pltpu.VMEM((_HTOK, _D), jnp.float32),
            pltpu.VMEM((_HTOK, _D), jnp.float32),
            pltpu.VMEM((_HTOK, _D), jnp.float32),
            pltpu.VMEM((_HTOK, _D), jnp.float32),
            pltpu.SemaphoreType.DMA,
            pltpu.SemaphoreType.DMA,
            pltpu.SemaphoreType.DMA,
            pltpu.SemaphoreType.DMA,
        ],
    )(_combine_body)
    out = combine(ys, d1v, d2v, w1v, w2v)

    return out, loss[0, 0]
